# baseline (speedup 1.0000x reference)
import functools

import jax
import jax.numpy as jnp
from jax import lax
from jax.experimental import pallas as pl
from jax.experimental.pallas import tpu as pltpu
from jax.experimental.pallas import tpu_sc as plsc

_S = 30.0
_M = 0.35

_NW = 32
_G = 128


def _sc_gather_build(batch):
    b_per_w = batch // _NW
    mesh = plsc.VectorSubcoreMesh(core_axis_name="c", subcore_axis_name="s")

    @functools.partial(
        pl.kernel, mesh=mesh,
        out_type=jax.ShapeDtypeStruct((batch, _G), jnp.float32),
        scratch_types=[
            pltpu.VMEM((b_per_w,), jnp.int32),
            pltpu.VMEM((b_per_w, _G), jnp.float32),
            pltpu.SemaphoreType.DMA,
        ],
    )
    def gather_k(flat_hbm, idx_hbm, out_hbm, idx_v, rows_v, sem):
        wid = lax.axis_index("s") * 2 + lax.axis_index("c")
        base = wid * b_per_w
        pltpu.sync_copy(idx_hbm.at[pl.ds(base, b_per_w)], idx_v)
        pltpu.async_copy(flat_hbm.at[idx_v], rows_v, sem).wait()
        pltpu.sync_copy(rows_v, out_hbm.at[pl.ds(base, b_per_w)])

    return gather_k


def _stream_body(n_cols, n_blocks, bc, x_ref, tg_ref, lane_ref, out_ref,
                 m_ref, s_ref):
    i = pl.program_id(0)

    @pl.when(i == 0)
    def _init():
        m_ref[...] = jnp.full_like(m_ref, -jnp.inf)
        s_ref[...] = jnp.zeros_like(s_ref)

    def update(x):
        m_old = m_ref[...]
        m_new = jnp.maximum(m_old, jnp.max(x, axis=1, keepdims=True))
        s_ref[...] = s_ref[...] * jnp.exp(_S * (m_old - m_new)) \
            + jnp.sum(jnp.exp(_S * (x - m_new)), axis=1, keepdims=True)
        m_ref[...] = m_new

    @pl.when(i < n_blocks - 1)
    def _main():
        update(x_ref[...])

    @pl.when(i == n_blocks - 1)
    def _tail():
        xb = x_ref[...]
        colids = lax.broadcasted_iota(jnp.int32, xb.shape, 1) + i * bc
        update(jnp.where(colids < n_cols, xb, -jnp.inf))

        tg = tg_ref[...]
        lane = lane_ref[...]
        lmatch = lax.broadcasted_iota(jnp.int32, tg.shape, 1) == lane
        t = jnp.sum(jnp.where(lmatch, tg, 0.0), axis=1, keepdims=True)
        m = m_ref[...]
        s = s_ref[...]
        e1 = jnp.exp(_S * (t - m))
        e2 = jnp.exp(_S * (t - _M - m))
        s_corr = jnp.maximum(s - e1, 0.0) + e2
        nll = jnp.log(s_corr) + _S * m - _S * (t - _M)
        out_ref[...] = jnp.sum(nll, axis=(0, 1), keepdims=True) / nll.shape[0]


@jax.jit
def kernel(input, label):
    n_rows, n_cols = input.shape
    lbl = label.astype(jnp.int32)

    n_groups = (n_rows * n_cols) // _G
    flat = input.reshape(n_groups, _G)
    fidx = jnp.arange(n_rows, dtype=jnp.int32) * n_cols + lbl
    idx = fidx // _G
    lane = fidx % _G
    tg = _sc_gather_build(n_rows)(flat, idx)

    bc = 1024
    n_blocks = pl.cdiv(n_cols, bc)
    body = lambda *refs: _stream_body(n_cols, n_blocks, bc, *refs)
    out = pl.pallas_call(
        body,
        grid=(n_blocks,),
        in_specs=[
            pl.BlockSpec((n_rows, bc), lambda i: (0, i)),
            pl.BlockSpec((n_rows, _G), lambda i: (0, 0)),
            pl.BlockSpec((n_rows, 1), lambda i: (0, 0)),
        ],
        out_specs=pl.BlockSpec((1, 1), lambda i: (0, 0)),
        out_shape=jax.ShapeDtypeStruct((1, 1), jnp.float32),
        scratch_shapes=[
            pltpu.VMEM((n_rows, 1), jnp.float32),
            pltpu.VMEM((n_rows, 1), jnp.float32),
        ],
    )(input, tg, lane.reshape(n_rows, 1))
    return out[0, 0]


# speedup vs baseline: 2.0526x; 2.0526x over previous
import jax
import jax.numpy as jnp
from jax import lax
from jax.experimental import pallas as pl
from jax.experimental.pallas import tpu as pltpu

_S = 30.0
_M = 0.35


def _stream_body(n_cols, n_blocks, bc, x_ref, lbl_ref, out_ref,
                 m_ref, s_ref, t_ref):
    i = pl.program_id(0)

    @pl.when(i == 0)
    def _init():
        m_ref[...] = jnp.full_like(m_ref, -jnp.inf)
        s_ref[...] = jnp.zeros_like(s_ref)
        t_ref[...] = jnp.zeros_like(t_ref)

    def update(x, xg):
        lbl_local = lbl_ref[...] - i * bc
        match = lax.broadcasted_iota(jnp.int32, x.shape, 1) == lbl_local
        t_ref[...] = t_ref[...] + jnp.sum(jnp.where(match, xg, 0.0),
                                          axis=1, keepdims=True)
        m_old = m_ref[...]
        m_new = jnp.maximum(m_old, jnp.max(x, axis=1, keepdims=True))
        s_ref[...] = s_ref[...] * jnp.exp(_S * (m_old - m_new)) \
            + jnp.sum(jnp.exp(_S * (x - m_new)), axis=1, keepdims=True)
        m_ref[...] = m_new

    @pl.when(i < n_blocks - 1)
    def _main():
        xb = x_ref[...]
        update(xb, xb)

    @pl.when(i == n_blocks - 1)
    def _tail():
        xb = x_ref[...]
        colids = lax.broadcasted_iota(jnp.int32, xb.shape, 1) + i * bc
        update(jnp.where(colids < n_cols, xb, -jnp.inf), xb)

        t = t_ref[...]
        m = m_ref[...]
        s = s_ref[...]
        e1 = jnp.exp(_S * (t - m))
        e2 = jnp.exp(_S * (t - _M - m))
        s_corr = jnp.maximum(s - e1, 0.0) + e2
        nll = jnp.log(s_corr) + _S * m - _S * (t - _M)
        out_ref[...] = jnp.sum(nll, axis=(0, 1), keepdims=True) / nll.shape[0]


@jax.jit
def kernel(input, label):
    n_rows, n_cols = input.shape
    lbl = label.astype(jnp.int32).reshape(n_rows, 1)

    bc = 1024
    n_blocks = pl.cdiv(n_cols, bc)
    body = lambda *refs: _stream_body(n_cols, n_blocks, bc, *refs)
    out = pl.pallas_call(
        body,
        grid=(n_blocks,),
        in_specs=[
            pl.BlockSpec((n_rows, bc), lambda i: (0, i)),
            pl.BlockSpec((n_rows, 1), lambda i: (0, 0)),
        ],
        out_specs=pl.BlockSpec((1, 1), lambda i: (0, 0)),
        out_shape=jax.ShapeDtypeStruct((1, 1), jnp.float32),
        scratch_shapes=[
            pltpu.VMEM((n_rows, 1), jnp.float32),
            pltpu.VMEM((n_rows, 1), jnp.float32),
            pltpu.VMEM((n_rows, 1), jnp.float32),
        ],
    )(input, lbl)
    return out[0, 0]
